# baseline (speedup 1.0000x reference)
import functools

import jax
import jax.numpy as jnp
from jax import lax
from jax.experimental import pallas as pl
from jax.experimental.pallas import tpu as pltpu
from jax.experimental.pallas import tpu_sc as plsc

N_NODES = 10000
N_EDGES = 160000
NUM_LAYERS = 3
IN_CH = 256
HID = 512
NUM_GRAPHS = 64
BN_EPS = 1e-5

NC = 2
NS = 16
FC = 128
BE = 125
HALF = 40
EPT = N_EDGES // NS
NI = EPT // BE
RPT = 624
TAIL = N_NODES - NS * RPT
TAIL0 = NS * RPT

M_BLK = 400
GRID_M = N_NODES // M_BLK


def _make_sc_agg(d_in):
    C = d_in // FC
    CPC = C // NC
    mesh = plsc.VectorSubcoreMesh(core_axis_name="c", subcore_axis_name="s")

    out_type = [jax.ShapeDtypeStruct((N_NODES, FC), jnp.float32) for _ in range(C)]
    scratch = [
        pltpu.VMEM_SHARED((N_NODES, FC), jnp.float32),
        pltpu.VMEM((BE, FC), jnp.float32),
        pltpu.VMEM((BE, FC), jnp.float32),
        pltpu.VMEM((HALF, BE), jnp.int32),
        pltpu.VMEM((HALF, BE), jnp.int32),
        pltpu.SemaphoreType.DMA,
        pltpu.SemaphoreType.DMA,
    ]

    @functools.partial(pl.kernel, mesh=mesh, out_type=out_type,
                       scratch_types=scratch)
    def sc_agg(*refs):
        h_refs = refs[:C]
        src_hbm = refs[C]
        dst_hbm = refs[C + 1]
        out_refs = refs[C + 2:C + 2 + C]
        agg, rows0, rows1, sidx, didx, sem0, sem1 = refs[C + 2 + C:]
        bufs = (rows0, rows1)
        sems = (sem0, sem1)

        c_idx = lax.axis_index("c")
        s_idx = lax.axis_index("s")

        row0 = pl.multiple_of(s_idx * RPT, 8)

        def per_core(c):
            def run():
                for cc in range(CPC):
                    ch = c * CPC + cc
                    pltpu.sync_copy(h_refs[ch].at[pl.ds(row0, RPT)],
                                    agg.at[pl.ds(row0, RPT)])

                    @pl.when(s_idx == 0)
                    def _():
                        pltpu.sync_copy(h_refs[ch].at[pl.ds(TAIL0, TAIL)],
                                        agg.at[pl.ds(TAIL0, TAIL)])

                    plsc.subcore_barrier()

                    def fire(j, b):
                        pltpu.make_async_copy(h_refs[ch].at[sidx.at[j]],
                                              bufs[b], sems[b]).start()

                    def drain(b):
                        pltpu.make_async_copy(h_refs[ch].at[sidx.at[0]],
                                              bufs[b], sems[b]).wait()

                    for half in range(NI // HALF):
                        pltpu.sync_copy(
                            src_hbm.at[s_idx, pl.ds(half * HALF, HALF)], sidx)
                        pltpu.sync_copy(
                            dst_hbm.at[s_idx, pl.ds(half * HALF, HALF)], didx)
                        fire(0, 0)
                        fire(1, 1)

                        def body(t, carry):
                            for b in range(2):
                                j = 2 * t + b
                                drain(b)
                                pltpu.sync_copy(bufs[b], agg.at[didx.at[j]],
                                                add=True)

                                @pl.when(t < HALF // 2 - 1)
                                def _():
                                    fire(j + 2, b)
                            return carry

                        lax.fori_loop(0, HALF // 2, body, 0)
                    plsc.subcore_barrier()
                    pltpu.sync_copy(agg.at[pl.ds(row0, RPT)],
                                    out_refs[ch].at[pl.ds(row0, RPT)])

                    @pl.when(s_idx == 0)
                    def _():
                        pltpu.sync_copy(agg.at[pl.ds(TAIL0, TAIL)],
                                        out_refs[ch].at[pl.ds(TAIL0, TAIL)])

                    plsc.subcore_barrier()
            return run

        for c in range(NC):
            pl.when(c_idx == c)(per_core(c))

    return sc_agg


def _make_mlp(d_in):
    C = d_in // FC

    def body(*refs):
        chunk_refs = refs[:C]
        w1, b1, w2, b2 = refs[C:C + 4]
        y_ref, stats_ref = refs[C + 4:]
        m = pl.program_id(0)

        zp = jnp.concatenate([r[...] for r in chunk_refs], axis=1)
        z1 = jnp.maximum(
            jnp.dot(zp, w1[...], preferred_element_type=jnp.float32) + b1[...],
            0.0)
        y = jnp.maximum(
            jnp.dot(z1, w2[...], preferred_element_type=jnp.float32) + b2[...],
            0.0)
        y_ref[...] = y

        s = jnp.sum(y, axis=0, keepdims=True)
        sq = jnp.sum(y * y, axis=0, keepdims=True)
        stats = jnp.concatenate([s, sq, jnp.zeros((6, HID), jnp.float32)],
                                axis=0)

        @pl.when(m == 0)
        def _():
            stats_ref[...] = jnp.zeros_like(stats_ref)

        stats_ref[...] += stats

    in_specs = (
        [pl.BlockSpec((M_BLK, FC), lambda m: (m, 0)) for _ in range(C)] + [
            pl.BlockSpec((d_in, HID), lambda m: (0, 0)),
            pl.BlockSpec((1, HID), lambda m: (0, 0)),
            pl.BlockSpec((HID, HID), lambda m: (0, 0)),
            pl.BlockSpec((1, HID), lambda m: (0, 0)),
        ])
    out_specs = [
        pl.BlockSpec((M_BLK, HID), lambda m: (m, 0)),
        pl.BlockSpec((8, HID), lambda m: (0, 0)),
    ]
    return pl.pallas_call(
        body,
        grid=(GRID_M,),
        in_specs=in_specs,
        out_specs=out_specs,
        out_shape=[
            jax.ShapeDtypeStruct((N_NODES, HID), jnp.float32),
            jax.ShapeDtypeStruct((8, HID), jnp.float32),
        ],
    )


def _norm_body(*refs):
    y_ref, stats_ref, gamma_ref, beta_ref = refs[:4]
    z_refs = refs[4:]
    n = jnp.float32(N_NODES)
    mean = stats_ref[0:1, :] / n
    var = stats_ref[1:2, :] / n - mean * mean
    inv = lax.rsqrt(var + BN_EPS)
    z = (y_ref[...] - mean) * (inv * gamma_ref[...]) + beta_ref[...]
    for k, zr in enumerate(z_refs):
        zr[...] = z[:, k * FC:(k + 1) * FC]


_norm = pl.pallas_call(
    _norm_body,
    grid=(GRID_M,),
    in_specs=[
        pl.BlockSpec((M_BLK, HID), lambda m: (m, 0)),
        pl.BlockSpec((8, HID), lambda m: (0, 0)),
        pl.BlockSpec((1, HID), lambda m: (0, 0)),
        pl.BlockSpec((1, HID), lambda m: (0, 0)),
    ],
    out_specs=[pl.BlockSpec((M_BLK, FC), lambda m: (m, 0))
               for _ in range(HID // FC)],
    out_shape=[jax.ShapeDtypeStruct((N_NODES, FC), jnp.float32)
               for _ in range(HID // FC)],
)


def _pool_body(*refs):
    chunk_refs = refs[:HID // FC]
    batch_ref = refs[HID // FC]
    pool_ref = refs[HID // FC + 1]
    m = pl.program_id(0)
    z = jnp.concatenate([r[...] for r in chunk_refs], axis=1)
    b = batch_ref[0, 0, :]
    gids = lax.broadcasted_iota(jnp.int32, (NUM_GRAPHS, M_BLK), 0)
    onehot = jnp.where(b[None, :] == gids, 1.0, 0.0)

    @pl.when(m == 0)
    def _():
        pool_ref[...] = jnp.zeros_like(pool_ref)

    pool_ref[...] += jnp.dot(onehot, z, preferred_element_type=jnp.float32)


_pool = pl.pallas_call(
    _pool_body,
    grid=(GRID_M,),
    in_specs=(
        [pl.BlockSpec((M_BLK, FC), lambda m: (m, 0))
         for _ in range(HID // FC)] +
        [pl.BlockSpec((1, 1, M_BLK), lambda m: (m, 0, 0))]),
    out_specs=[pl.BlockSpec((NUM_GRAPHS, HID), lambda m: (0, 0))],
    out_shape=[jax.ShapeDtypeStruct((NUM_GRAPHS, HID), jnp.float32)],
)

_sc_agg_256 = _make_sc_agg(IN_CH)
_sc_agg_512 = _make_sc_agg(HID)
_mlp_256 = _make_mlp(IN_CH)
_mlp_512 = _make_mlp(HID)


def kernel(x, edge_index, batch, params):
    src = edge_index[0].reshape(NS, NI, BE)
    dst = edge_index[1].reshape(NS, NI, BE)
    batch3 = batch.reshape(GRID_M, 1, M_BLK)

    h_chunks = [x[:, k * FC:(k + 1) * FC] for k in range(IN_CH // FC)]
    pools = []
    xs = jnp.zeros((N_NODES, NUM_LAYERS * HID), jnp.float32)
    for i, p in enumerate(params):
        sc_agg = _sc_agg_256 if i == 0 else _sc_agg_512
        mlp = _mlp_256 if i == 0 else _mlp_512
        zpre_chunks = sc_agg(*h_chunks, src, dst)
        y, stats = mlp(*zpre_chunks,
                       p["W1"], p["b1"].reshape(1, HID),
                       p["W2"], p["b2"].reshape(1, HID))
        z_chunks = _norm(y, stats,
                         p["gamma"].reshape(1, HID),
                         p["beta"].reshape(1, HID))
        (pool,) = _pool(*z_chunks, batch3)
        pools.append(pool)
        for k, ch in enumerate(z_chunks):
            xs = lax.dynamic_update_slice(xs, ch, (0, i * HID + k * FC))
        h_chunks = z_chunks

    return jnp.concatenate(pools, axis=1), xs


# speedup vs baseline: 1.1117x; 1.1117x over previous
import functools

import jax
import jax.numpy as jnp
from jax import lax
from jax.experimental import pallas as pl
from jax.experimental.pallas import tpu as pltpu
from jax.experimental.pallas import tpu_sc as plsc

N_NODES = 10000
N_EDGES = 160000
NUM_LAYERS = 3
IN_CH = 256
HID = 512
NUM_GRAPHS = 64
BN_EPS = 1e-5

NC = 2
NS = 16
FC = 128
BE = 125
HALF = 40
EPT = N_EDGES // NS
NI = EPT // BE
RPT = 624
TAIL = N_NODES - NS * RPT
TAIL0 = NS * RPT

M_BLK = 1000
GRID_M = N_NODES // M_BLK


def _make_sc_agg(d_in):
    C = d_in // FC
    CPC = C // NC
    mesh = plsc.VectorSubcoreMesh(core_axis_name="c", subcore_axis_name="s")

    out_type = [jax.ShapeDtypeStruct((N_NODES, FC), jnp.float32) for _ in range(C)]
    scratch = [
        pltpu.VMEM_SHARED((N_NODES, FC), jnp.float32),
        pltpu.VMEM((BE, FC), jnp.float32),
        pltpu.VMEM((BE, FC), jnp.float32),
        pltpu.VMEM((HALF, BE), jnp.int32),
        pltpu.VMEM((HALF, BE), jnp.int32),
        pltpu.SemaphoreType.DMA,
        pltpu.SemaphoreType.DMA,
    ]

    @functools.partial(pl.kernel, mesh=mesh, out_type=out_type,
                       scratch_types=scratch)
    def sc_agg(*refs):
        h_refs = refs[:C]
        src_hbm = refs[C]
        dst_hbm = refs[C + 1]
        out_refs = refs[C + 2:C + 2 + C]
        agg, rows0, rows1, sidx, didx, sem0, sem1 = refs[C + 2 + C:]
        bufs = (rows0, rows1)
        sems = (sem0, sem1)

        c_idx = lax.axis_index("c")
        s_idx = lax.axis_index("s")

        row0 = pl.multiple_of(s_idx * RPT, 8)

        def per_core(c):
            def run():
                for cc in range(CPC):
                    ch = c * CPC + cc
                    pltpu.sync_copy(h_refs[ch].at[pl.ds(row0, RPT)],
                                    agg.at[pl.ds(row0, RPT)])

                    @pl.when(s_idx == 0)
                    def _():
                        pltpu.sync_copy(h_refs[ch].at[pl.ds(TAIL0, TAIL)],
                                        agg.at[pl.ds(TAIL0, TAIL)])

                    plsc.subcore_barrier()

                    def fire(j, b):
                        pltpu.make_async_copy(h_refs[ch].at[sidx.at[j]],
                                              bufs[b], sems[b]).start()

                    def drain(b):
                        pltpu.make_async_copy(h_refs[ch].at[sidx.at[0]],
                                              bufs[b], sems[b]).wait()

                    for half in range(NI // HALF):
                        pltpu.sync_copy(
                            src_hbm.at[s_idx, pl.ds(half * HALF, HALF)], sidx)
                        pltpu.sync_copy(
                            dst_hbm.at[s_idx, pl.ds(half * HALF, HALF)], didx)
                        fire(0, 0)
                        fire(1, 1)

                        def body(t, carry):
                            for b in range(2):
                                j = 2 * t + b
                                drain(b)
                                pltpu.sync_copy(bufs[b], agg.at[didx.at[j]],
                                                add=True)

                                @pl.when(t < HALF // 2 - 1)
                                def _():
                                    fire(j + 2, b)
                            return carry

                        lax.fori_loop(0, HALF // 2, body, 0)
                    plsc.subcore_barrier()
                    pltpu.sync_copy(agg.at[pl.ds(row0, RPT)],
                                    out_refs[ch].at[pl.ds(row0, RPT)])

                    @pl.when(s_idx == 0)
                    def _():
                        pltpu.sync_copy(agg.at[pl.ds(TAIL0, TAIL)],
                                        out_refs[ch].at[pl.ds(TAIL0, TAIL)])

                    plsc.subcore_barrier()
            return run

        for c in range(NC):
            pl.when(c_idx == c)(per_core(c))

    return sc_agg


def _make_mlp(d_in):
    C = d_in // FC

    def body(*refs):
        chunk_refs = refs[:C]
        w1, b1, w2, b2 = refs[C:C + 4]
        y_ref, stats_ref = refs[C + 4:]
        m = pl.program_id(0)

        zp = jnp.concatenate([r[...] for r in chunk_refs], axis=1)
        z1 = jnp.maximum(
            jnp.dot(zp, w1[...], preferred_element_type=jnp.float32) + b1[...],
            0.0)
        y = jnp.maximum(
            jnp.dot(z1, w2[...], preferred_element_type=jnp.float32) + b2[...],
            0.0)
        y_ref[...] = y

        s = jnp.sum(y, axis=0, keepdims=True)
        sq = jnp.sum(y * y, axis=0, keepdims=True)
        stats = jnp.concatenate([s, sq, jnp.zeros((6, HID), jnp.float32)],
                                axis=0)

        @pl.when(m == 0)
        def _():
            stats_ref[...] = jnp.zeros_like(stats_ref)

        stats_ref[...] += stats

    in_specs = (
        [pl.BlockSpec((M_BLK, FC), lambda m: (m, 0)) for _ in range(C)] + [
            pl.BlockSpec((d_in, HID), lambda m: (0, 0)),
            pl.BlockSpec((1, HID), lambda m: (0, 0)),
            pl.BlockSpec((HID, HID), lambda m: (0, 0)),
            pl.BlockSpec((1, HID), lambda m: (0, 0)),
        ])
    out_specs = [
        pl.BlockSpec((M_BLK, HID), lambda m: (m, 0)),
        pl.BlockSpec((8, HID), lambda m: (0, 0)),
    ]
    return pl.pallas_call(
        body,
        grid=(GRID_M,),
        in_specs=in_specs,
        out_specs=out_specs,
        out_shape=[
            jax.ShapeDtypeStruct((N_NODES, HID), jnp.float32),
            jax.ShapeDtypeStruct((8, HID), jnp.float32),
        ],
    )


def _norm_pool_body(*refs):
    y_ref, stats_ref, gamma_ref, beta_ref, batch_ref = refs[:5]
    z_refs = refs[5:5 + HID // FC]
    pool_ref = refs[5 + HID // FC]
    m = pl.program_id(0)
    n = jnp.float32(N_NODES)
    mean = stats_ref[0:1, :] / n
    var = stats_ref[1:2, :] / n - mean * mean
    inv = lax.rsqrt(var + BN_EPS)
    z = (y_ref[...] - mean) * (inv * gamma_ref[...]) + beta_ref[...]
    for k, zr in enumerate(z_refs):
        zr[...] = z[:, k * FC:(k + 1) * FC]

    b = batch_ref[0, 0, :]
    gids = lax.broadcasted_iota(jnp.int32, (NUM_GRAPHS, M_BLK), 0)
    onehot = jnp.where(b[None, :] == gids, 1.0, 0.0)

    @pl.when(m == 0)
    def _():
        pool_ref[...] = jnp.zeros_like(pool_ref)

    pool_ref[...] += jnp.dot(onehot, z, preferred_element_type=jnp.float32)


_norm_pool = pl.pallas_call(
    _norm_pool_body,
    grid=(GRID_M,),
    in_specs=[
        pl.BlockSpec((M_BLK, HID), lambda m: (m, 0)),
        pl.BlockSpec((8, HID), lambda m: (0, 0)),
        pl.BlockSpec((1, HID), lambda m: (0, 0)),
        pl.BlockSpec((1, HID), lambda m: (0, 0)),
        pl.BlockSpec((1, 1, M_BLK), lambda m: (m, 0, 0)),
    ],
    out_specs=(
        [pl.BlockSpec((M_BLK, FC), lambda m: (m, 0))
         for _ in range(HID // FC)] +
        [pl.BlockSpec((NUM_GRAPHS, HID), lambda m: (0, 0))]),
    out_shape=(
        [jax.ShapeDtypeStruct((N_NODES, FC), jnp.float32)
         for _ in range(HID // FC)] +
        [jax.ShapeDtypeStruct((NUM_GRAPHS, HID), jnp.float32)]),
)

_sc_agg_256 = _make_sc_agg(IN_CH)
_sc_agg_512 = _make_sc_agg(HID)
_mlp_256 = _make_mlp(IN_CH)
_mlp_512 = _make_mlp(HID)


def kernel(x, edge_index, batch, params):
    src = edge_index[0].reshape(NS, NI, BE)
    dst = edge_index[1].reshape(NS, NI, BE)
    batch3 = batch.reshape(GRID_M, 1, M_BLK)

    h_chunks = [x[:, k * FC:(k + 1) * FC] for k in range(IN_CH // FC)]
    pools = []
    zs = []
    for i, p in enumerate(params):
        sc_agg = _sc_agg_256 if i == 0 else _sc_agg_512
        mlp = _mlp_256 if i == 0 else _mlp_512
        zpre_chunks = sc_agg(*h_chunks, src, dst)
        y, stats = mlp(*zpre_chunks,
                       p["W1"], p["b1"].reshape(1, HID),
                       p["W2"], p["b2"].reshape(1, HID))
        out = _norm_pool(y, stats,
                         p["gamma"].reshape(1, HID),
                         p["beta"].reshape(1, HID), batch3)
        z_chunks, pool = list(out[:HID // FC]), out[HID // FC]
        zs.extend(z_chunks)
        pools.append(pool)
        h_chunks = z_chunks

    return jnp.concatenate(pools, axis=1), jnp.concatenate(zs, axis=1)


# speedup vs baseline: 1.1265x; 1.0133x over previous
import functools

import jax
import jax.numpy as jnp
from jax import lax
from jax.experimental import pallas as pl
from jax.experimental.pallas import tpu as pltpu
from jax.experimental.pallas import tpu_sc as plsc

N_NODES = 10000
N_EDGES = 160000
NUM_LAYERS = 3
IN_CH = 256
HID = 512
NUM_GRAPHS = 64
BN_EPS = 1e-5

NC = 2
NS = 16
FC = 128
BE = 125
HALF = 40
EPT = N_EDGES // NS
NI = EPT // BE
RPT = 624
TAIL = N_NODES - NS * RPT
TAIL0 = NS * RPT

M_BLK = 2000
GRID_M = N_NODES // M_BLK


def _make_sc_agg(d_in):
    C = d_in // FC
    CPC = C // NC
    mesh = plsc.VectorSubcoreMesh(core_axis_name="c", subcore_axis_name="s")

    out_type = [jax.ShapeDtypeStruct((N_NODES, FC), jnp.float32) for _ in range(C)]
    scratch = [
        pltpu.VMEM_SHARED((N_NODES, FC), jnp.float32),
        pltpu.VMEM((BE, FC), jnp.float32),
        pltpu.VMEM((BE, FC), jnp.float32),
        pltpu.VMEM((HALF, BE), jnp.int32),
        pltpu.VMEM((HALF, BE), jnp.int32),
        pltpu.SemaphoreType.DMA,
        pltpu.SemaphoreType.DMA,
    ]

    @functools.partial(pl.kernel, mesh=mesh, out_type=out_type,
                       scratch_types=scratch)
    def sc_agg(*refs):
        h_refs = refs[:C]
        src_hbm = refs[C]
        dst_hbm = refs[C + 1]
        out_refs = refs[C + 2:C + 2 + C]
        agg, rows0, rows1, sidx, didx, sem0, sem1 = refs[C + 2 + C:]
        bufs = (rows0, rows1)
        sems = (sem0, sem1)

        c_idx = lax.axis_index("c")
        s_idx = lax.axis_index("s")

        row0 = pl.multiple_of(s_idx * RPT, 8)

        def per_core(c):
            def run():
                for cc in range(CPC):
                    ch = c * CPC + cc
                    pltpu.sync_copy(h_refs[ch].at[pl.ds(row0, RPT)],
                                    agg.at[pl.ds(row0, RPT)])

                    @pl.when(s_idx == 0)
                    def _():
                        pltpu.sync_copy(h_refs[ch].at[pl.ds(TAIL0, TAIL)],
                                        agg.at[pl.ds(TAIL0, TAIL)])

                    plsc.subcore_barrier()

                    def fire(j, b):
                        pltpu.make_async_copy(h_refs[ch].at[sidx.at[j]],
                                              bufs[b], sems[b]).start()

                    def drain(b):
                        pltpu.make_async_copy(h_refs[ch].at[sidx.at[0]],
                                              bufs[b], sems[b]).wait()

                    for half in range(NI // HALF):
                        pltpu.sync_copy(
                            src_hbm.at[s_idx, pl.ds(half * HALF, HALF)], sidx)
                        pltpu.sync_copy(
                            dst_hbm.at[s_idx, pl.ds(half * HALF, HALF)], didx)
                        fire(0, 0)
                        fire(1, 1)

                        def body(t, carry):
                            for b in range(2):
                                j = 2 * t + b
                                drain(b)
                                pltpu.sync_copy(bufs[b], agg.at[didx.at[j]],
                                                add=True)

                                @pl.when(t < HALF // 2 - 1)
                                def _():
                                    fire(j + 2, b)
                            return carry

                        lax.fori_loop(0, HALF // 2, body, 0)
                    plsc.subcore_barrier()
                    pltpu.sync_copy(agg.at[pl.ds(row0, RPT)],
                                    out_refs[ch].at[pl.ds(row0, RPT)])

                    @pl.when(s_idx == 0)
                    def _():
                        pltpu.sync_copy(agg.at[pl.ds(TAIL0, TAIL)],
                                        out_refs[ch].at[pl.ds(TAIL0, TAIL)])

                    plsc.subcore_barrier()
            return run

        for c in range(NC):
            pl.when(c_idx == c)(per_core(c))

    return sc_agg


def _make_mlp(d_in):
    C = d_in // FC

    def body(*refs):
        chunk_refs = refs[:C]
        w1, b1, w2, b2 = refs[C:C + 4]
        y_ref, stats_ref = refs[C + 4:]
        m = pl.program_id(0)

        zp = jnp.concatenate([r[...] for r in chunk_refs], axis=1)
        z1 = jnp.maximum(
            jnp.dot(zp, w1[...], preferred_element_type=jnp.float32) + b1[...],
            0.0)
        y = jnp.maximum(
            jnp.dot(z1, w2[...], preferred_element_type=jnp.float32) + b2[...],
            0.0)
        y_ref[...] = y

        s = jnp.sum(y, axis=0, keepdims=True)
        sq = jnp.sum(y * y, axis=0, keepdims=True)
        stats = jnp.concatenate([s, sq, jnp.zeros((6, HID), jnp.float32)],
                                axis=0)

        @pl.when(m == 0)
        def _():
            stats_ref[...] = jnp.zeros_like(stats_ref)

        stats_ref[...] += stats

    in_specs = (
        [pl.BlockSpec((M_BLK, FC), lambda m: (m, 0)) for _ in range(C)] + [
            pl.BlockSpec((d_in, HID), lambda m: (0, 0)),
            pl.BlockSpec((1, HID), lambda m: (0, 0)),
            pl.BlockSpec((HID, HID), lambda m: (0, 0)),
            pl.BlockSpec((1, HID), lambda m: (0, 0)),
        ])
    out_specs = [
        pl.BlockSpec((M_BLK, HID), lambda m: (m, 0)),
        pl.BlockSpec((8, HID), lambda m: (0, 0)),
    ]
    return pl.pallas_call(
        body,
        grid=(GRID_M,),
        in_specs=in_specs,
        out_specs=out_specs,
        out_shape=[
            jax.ShapeDtypeStruct((N_NODES, HID), jnp.float32),
            jax.ShapeDtypeStruct((8, HID), jnp.float32),
        ],
    )


def _norm_pool_body(*refs):
    y_ref, stats_ref, gamma_ref, beta_ref, batch_ref = refs[:5]
    z_refs = refs[5:5 + HID // FC]
    pool_ref = refs[5 + HID // FC]
    m = pl.program_id(0)
    n = jnp.float32(N_NODES)
    mean = stats_ref[0:1, :] / n
    var = stats_ref[1:2, :] / n - mean * mean
    inv = lax.rsqrt(var + BN_EPS)
    z = (y_ref[...] - mean) * (inv * gamma_ref[...]) + beta_ref[...]
    for k, zr in enumerate(z_refs):
        zr[...] = z[:, k * FC:(k + 1) * FC]

    b = batch_ref[0, 0, :]
    gids = lax.broadcasted_iota(jnp.int32, (NUM_GRAPHS, M_BLK), 0)
    onehot = jnp.where(b[None, :] == gids, 1.0, 0.0)

    @pl.when(m == 0)
    def _():
        pool_ref[...] = jnp.zeros_like(pool_ref)

    pool_ref[...] += jnp.dot(onehot, z, preferred_element_type=jnp.float32)


_norm_pool = pl.pallas_call(
    _norm_pool_body,
    grid=(GRID_M,),
    in_specs=[
        pl.BlockSpec((M_BLK, HID), lambda m: (m, 0)),
        pl.BlockSpec((8, HID), lambda m: (0, 0)),
        pl.BlockSpec((1, HID), lambda m: (0, 0)),
        pl.BlockSpec((1, HID), lambda m: (0, 0)),
        pl.BlockSpec((1, 1, M_BLK), lambda m: (m, 0, 0)),
    ],
    out_specs=(
        [pl.BlockSpec((M_BLK, FC), lambda m: (m, 0))
         for _ in range(HID // FC)] +
        [pl.BlockSpec((NUM_GRAPHS, HID), lambda m: (0, 0))]),
    out_shape=(
        [jax.ShapeDtypeStruct((N_NODES, FC), jnp.float32)
         for _ in range(HID // FC)] +
        [jax.ShapeDtypeStruct((NUM_GRAPHS, HID), jnp.float32)]),
)

_sc_agg_256 = _make_sc_agg(IN_CH)
_sc_agg_512 = _make_sc_agg(HID)
_mlp_256 = _make_mlp(IN_CH)
_mlp_512 = _make_mlp(HID)


def kernel(x, edge_index, batch, params):
    src = edge_index[0].reshape(NS, NI, BE)
    dst = edge_index[1].reshape(NS, NI, BE)
    batch3 = batch.reshape(GRID_M, 1, M_BLK)

    h_chunks = [x[:, k * FC:(k + 1) * FC] for k in range(IN_CH // FC)]
    pools = []
    zs = []
    for i, p in enumerate(params):
        sc_agg = _sc_agg_256 if i == 0 else _sc_agg_512
        mlp = _mlp_256 if i == 0 else _mlp_512
        zpre_chunks = sc_agg(*h_chunks, src, dst)
        y, stats = mlp(*zpre_chunks,
                       p["W1"], p["b1"].reshape(1, HID),
                       p["W2"], p["b2"].reshape(1, HID))
        out = _norm_pool(y, stats,
                         p["gamma"].reshape(1, HID),
                         p["beta"].reshape(1, HID), batch3)
        z_chunks, pool = list(out[:HID // FC]), out[HID // FC]
        zs.extend(z_chunks)
        pools.append(pool)
        h_chunks = z_chunks

    return jnp.concatenate(pools, axis=1), jnp.concatenate(zs, axis=1)
